# baseline (speedup 1.0000x reference)
import functools

import jax
import jax.numpy as jnp
from jax import lax
from jax.experimental import pallas as pl
from jax.experimental.pallas import tpu as pltpu
from jax.experimental.pallas import tpu_sc as plsc

_VOCAB = 1000000
_EMBED = 64
_CTX = 200
_HID = 120

_TILE = 32768
_NT = (_VOCAB + _TILE - 1) // _TILE
_VPAD = _NT * _TILE

_LANES = 128
_TAIL0 = (_VOCAB // _LANES) * _LANES
_TAILW = _VOCAB - _TAIL0

_NC = 2
_NS = 16
_RPW = 8
_ACTIVE = _CTX // _RPW


@functools.partial(
    pl.kernel,
    out_type=jax.ShapeDtypeStruct((_CTX, _EMBED, _LANES), jnp.float32),
    mesh=plsc.VectorSubcoreMesh(core_axis_name="c", subcore_axis_name="s"),
    scratch_types=[
        pltpu.VMEM((16,), jnp.int32),
        pltpu.VMEM((_RPW, _EMBED, _LANES), jnp.float32),
        pltpu.SemaphoreType.DMA,
    ],
)
def _sc_gather(x_hbm, embt_hbm, out_hbm, idx_v, tbuf, sem):
    wid = lax.axis_index("s") * _NC + lax.axis_index("c")

    @pl.when(wid < _ACTIVE)
    def _():
        base = pl.multiple_of(wid * _RPW, _RPW)
        pltpu.sync_copy(x_hbm.at[pl.ds(base, _RPW)], idx_v.at[pl.ds(0, _RPW)])
        idx_reg = idx_v[...]
        copies = []
        for j in range(_RPW):
            win = jnp.minimum(idx_reg[j] // _LANES, _TAIL0 // _LANES - 1)
            start = pl.multiple_of(win * _LANES, _LANES)
            copies.append(pltpu.async_copy(
                embt_hbm.at[:, pl.ds(start, _LANES)], tbuf.at[j], sem))
        for c in copies:
            c.wait()
        pltpu.sync_copy(tbuf, out_hbm.at[pl.ds(base, _RPW)])


def _mlp_body(win_ref, l_ref, w1_ref, b1_ref, w2t_ref, b2_ref,
              logits_ref, logz_ref, h_s, m_s, s_s):
    i = pl.program_id(0)

    @pl.when(i == 0)
    def _():
        lanes = lax.broadcasted_iota(jnp.int32, (_CTX, _EMBED, _LANES), 2)
        sel = jnp.where(lanes == l_ref[...].reshape(_CTX, 1, 1),
                        win_ref[...], 0.0)
        wm = sel.reshape(_CTX * _EMBED, _LANES)
        h128 = lax.dot_general(
            w1_ref[...], wm, (((1,), (0,)), ((), ())),
            preferred_element_type=jnp.float32)
        hv = jnp.sum(h128, axis=1, keepdims=True)
        h_s[...] = jnp.maximum(hv + b1_ref[...], 0.0)
        m_s[...] = jnp.full((1, 128), -jnp.inf, jnp.float32)
        s_s[...] = jnp.zeros((1, 128), jnp.float32)

    logits = lax.dot_general(
        h_s[...].astype(jnp.bfloat16), w2t_ref[...].astype(jnp.bfloat16),
        (((0,), (0,)), ((), ())),
        preferred_element_type=jnp.float32)
    logits = logits + b2_ref[...].reshape(1, _TILE)
    logits_ref[...] = logits.reshape(_TILE)

    lane = lax.broadcasted_iota(jnp.int32, (1, _TILE), 1)
    valid = lane < (_VOCAB - i * _TILE)
    lm = jnp.where(valid, logits, -jnp.inf)

    t_max = jnp.max(lm, axis=1, keepdims=True)
    m_old = m_s[0:1, 0:1]
    s_old = s_s[0:1, 0:1]
    m_new = jnp.maximum(m_old, t_max)
    t_sum = jnp.sum(jnp.exp(lm - m_new), axis=1, keepdims=True)
    s_new = s_old * jnp.exp(m_old - m_new) + t_sum
    m_s[0:1, 0:1] = m_new
    s_s[0:1, 0:1] = s_new

    @pl.when(i == _NT - 1)
    def _():
        logz_ref[...] = jnp.broadcast_to(m_new + jnp.log(s_new), (1, 128))


_SUBBLK = 32768
_SUBGRID = (_VOCAB + _SUBBLK - 1) // _SUBBLK


def _sub_body(logits_ref, logz_ref, out_ref):
    out_ref[...] = (logits_ref[...] - logz_ref[0:1, 0:1]).reshape(1, _SUBBLK)


def kernel(x, emb, W1, b1, W2, b2):
    embt = jnp.swapaxes(emb, 0, 1)
    xi = x.astype(jnp.int32)
    win = _sc_gather(xi, embt)
    cond = xi >= _TAIL0
    tail = lax.slice_in_dim(emb, _TAIL0, _VOCAB, axis=0)
    tv = jnp.take(tail, jnp.clip(xi - _TAIL0, 0, _TAILW - 1), axis=0)
    lane0 = jnp.where(cond[:, None], tv, win[:, :, 0])
    win = lax.dynamic_update_slice_in_dim(win, lane0[:, :, None], 0, axis=2)
    l_col = jnp.where(cond, 0, xi % _LANES).reshape(_CTX, 1)
    b1c = b1.reshape(_HID, 1)
    w2t = jnp.swapaxes(W2, 0, 1)

    logits, logz = pl.pallas_call(
        _mlp_body,
        grid=(_NT,),
        in_specs=[
            pl.BlockSpec((_CTX, _EMBED, _LANES), lambda i: (0, 0, 0)),
            pl.BlockSpec((_CTX, 1), lambda i: (0, 0)),
            pl.BlockSpec((_HID, _CTX * _EMBED), lambda i: (0, 0)),
            pl.BlockSpec((_HID, 1), lambda i: (0, 0)),
            pl.BlockSpec((_HID, _TILE), lambda i: (0, i)),
            pl.BlockSpec((_TILE,), lambda i: (i,)),
        ],
        out_specs=[
            pl.BlockSpec((_TILE,), lambda i: (i,)),
            pl.BlockSpec((1, 128), lambda i: (0, 0)),
        ],
        out_shape=[
            jax.ShapeDtypeStruct((_VPAD,), jnp.float32),
            jax.ShapeDtypeStruct((1, 128), jnp.float32),
        ],
        scratch_shapes=[
            pltpu.VMEM((_HID, 1), jnp.float32),
            pltpu.VMEM((1, 128), jnp.float32),
            pltpu.VMEM((1, 128), jnp.float32),
        ],
        compiler_params=pltpu.CompilerParams(
            dimension_semantics=("arbitrary",)),
    )(win, l_col, W1, b1c, w2t, b2)

    log_prob = pl.pallas_call(
        _sub_body,
        grid=(_SUBGRID,),
        in_specs=[
            pl.BlockSpec((_SUBBLK,), lambda i: (i,)),
            pl.BlockSpec((1, 128), lambda i: (0, 0)),
        ],
        out_specs=pl.BlockSpec((1, _SUBBLK), lambda i: (0, i)),
        out_shape=jax.ShapeDtypeStruct((1, _VOCAB), jnp.float32),
        compiler_params=pltpu.CompilerParams(
            dimension_semantics=("arbitrary",)),
    )(logits, logz)

    return log_prob


# speedup vs baseline: 1.1492x; 1.1492x over previous
import functools

import jax
import jax.numpy as jnp
from jax import lax
from jax.experimental import pallas as pl
from jax.experimental.pallas import tpu as pltpu
from jax.experimental.pallas import tpu_sc as plsc

_VOCAB = 1000000
_EMBED = 64
_CTX = 200
_HID = 120

_TILE = 32768
_NT = (_VOCAB + _TILE - 1) // _TILE
_VPAD = _NT * _TILE

_LANES = 128
_TAIL0 = (_VOCAB // _LANES) * _LANES
_TAILW = _VOCAB - _TAIL0

_NC = 2
_NS = 16
_RPW = 8
_ACTIVE = _CTX // _RPW


@functools.partial(
    pl.kernel,
    out_type=jax.ShapeDtypeStruct((_CTX, _EMBED, _LANES), jnp.float32),
    mesh=plsc.VectorSubcoreMesh(core_axis_name="c", subcore_axis_name="s"),
    scratch_types=[
        pltpu.VMEM((16,), jnp.int32),
        pltpu.VMEM((_RPW, _EMBED, _LANES), jnp.float32),
        pltpu.SemaphoreType.DMA,
    ],
)
def _sc_gather(x_hbm, embt_hbm, out_hbm, idx_v, tbuf, sem):
    wid = lax.axis_index("s") * _NC + lax.axis_index("c")

    @pl.when(wid < _ACTIVE)
    def _():
        base = pl.multiple_of(wid * _RPW, _RPW)
        pltpu.sync_copy(x_hbm.at[pl.ds(base, _RPW)], idx_v.at[pl.ds(0, _RPW)])
        idx_reg = idx_v[...]
        copies = []
        for j in range(_RPW):
            win = jnp.minimum(idx_reg[j] // _LANES, _TAIL0 // _LANES - 1)
            start = pl.multiple_of(win * _LANES, _LANES)
            copies.append(pltpu.async_copy(
                embt_hbm.at[:, pl.ds(start, _LANES)], tbuf.at[j], sem))
        for c in copies:
            c.wait()
        pltpu.sync_copy(tbuf, out_hbm.at[pl.ds(base, _RPW)])


def _mlp_body(win_ref, l_ref, tv_ref, w1_ref, b1_ref, w2t_ref, b2_ref,
              logits_ref, logz_ref, h_s, m_s, s_s):
    i = pl.program_id(0)

    @pl.when(i == 0)
    def _():
        lanes = lax.broadcasted_iota(jnp.int32, (_CTX, _EMBED, _LANES), 2)
        l3 = l_ref[...].reshape(_CTX, 1, 1)
        sel = jnp.where(lanes == l3, win_ref[...], 0.0)
        sel = jnp.where((l3 == _LANES) & (lanes == 0),
                        tv_ref[...][:, :, None], sel)
        wm = sel.reshape(_CTX * _EMBED, _LANES)
        h128 = lax.dot_general(
            w1_ref[...], wm, (((1,), (0,)), ((), ())),
            preferred_element_type=jnp.float32)
        hv = jnp.sum(h128, axis=1, keepdims=True)
        h_s[...] = jnp.maximum(hv + b1_ref[...], 0.0)
        m_s[...] = jnp.full((1, 128), -jnp.inf, jnp.float32)
        s_s[...] = jnp.zeros((1, 128), jnp.float32)

    logits = lax.dot_general(
        h_s[...].astype(jnp.bfloat16), w2t_ref[...].astype(jnp.bfloat16),
        (((0,), (0,)), ((), ())),
        preferred_element_type=jnp.float32)
    logits = logits + b2_ref[...].reshape(1, _TILE)
    logits_ref[...] = logits.reshape(_TILE)

    lane = lax.broadcasted_iota(jnp.int32, (1, _TILE), 1)
    valid = lane < (_VOCAB - i * _TILE)
    lm = jnp.where(valid, logits, -jnp.inf)

    t_max = jnp.max(lm, axis=1, keepdims=True)
    m_old = m_s[0:1, 0:1]
    s_old = s_s[0:1, 0:1]
    m_new = jnp.maximum(m_old, t_max)
    t_sum = jnp.sum(jnp.exp(lm - m_new), axis=1, keepdims=True)
    s_new = s_old * jnp.exp(m_old - m_new) + t_sum
    m_s[0:1, 0:1] = m_new
    s_s[0:1, 0:1] = s_new

    @pl.when(i == _NT - 1)
    def _():
        logz_ref[...] = jnp.broadcast_to(m_new + jnp.log(s_new), (1, 128))


_SUBBLK = 32768
_SUBGRID = (_VOCAB + _SUBBLK - 1) // _SUBBLK


def _sub_body(logits_ref, logz_ref, out_ref):
    out_ref[...] = (logits_ref[...] - logz_ref[0:1, 0:1]).reshape(1, _SUBBLK)


def kernel(x, emb, W1, b1, W2, b2):
    embt = jnp.swapaxes(emb, 0, 1)
    xi = x.astype(jnp.int32)
    win = _sc_gather(xi, embt)
    cond = xi >= _TAIL0
    tail = lax.slice_in_dim(emb, _TAIL0, _VOCAB, axis=0)
    tv = jnp.take(tail, jnp.clip(xi - _TAIL0, 0, _TAILW - 1), axis=0)
    l_col = jnp.where(cond, _LANES, xi % _LANES).reshape(_CTX, 1)
    b1c = b1.reshape(_HID, 1)
    w2t = jnp.swapaxes(W2, 0, 1)

    logits, logz = pl.pallas_call(
        _mlp_body,
        grid=(_NT,),
        in_specs=[
            pl.BlockSpec((_CTX, _EMBED, _LANES), lambda i: (0, 0, 0)),
            pl.BlockSpec((_CTX, 1), lambda i: (0, 0)),
            pl.BlockSpec((_CTX, _EMBED), lambda i: (0, 0)),
            pl.BlockSpec((_HID, _CTX * _EMBED), lambda i: (0, 0)),
            pl.BlockSpec((_HID, 1), lambda i: (0, 0)),
            pl.BlockSpec((_HID, _TILE), lambda i: (0, i)),
            pl.BlockSpec((_TILE,), lambda i: (i,)),
        ],
        out_specs=[
            pl.BlockSpec((_TILE,), lambda i: (i,)),
            pl.BlockSpec((1, 128), lambda i: (0, 0)),
        ],
        out_shape=[
            jax.ShapeDtypeStruct((_VPAD,), jnp.float32),
            jax.ShapeDtypeStruct((1, 128), jnp.float32),
        ],
        scratch_shapes=[
            pltpu.VMEM((_HID, 1), jnp.float32),
            pltpu.VMEM((1, 128), jnp.float32),
            pltpu.VMEM((1, 128), jnp.float32),
        ],
        compiler_params=pltpu.CompilerParams(
            dimension_semantics=("arbitrary",)),
    )(win, l_col, tv, W1, b1c, w2t, b2)

    log_prob = pl.pallas_call(
        _sub_body,
        grid=(_SUBGRID,),
        in_specs=[
            pl.BlockSpec((_SUBBLK,), lambda i: (i,)),
            pl.BlockSpec((1, 128), lambda i: (0, 0)),
        ],
        out_specs=pl.BlockSpec((1, _SUBBLK), lambda i: (0, i)),
        out_shape=jax.ShapeDtypeStruct((1, _VOCAB), jnp.float32),
        compiler_params=pltpu.CompilerParams(
            dimension_semantics=("arbitrary",)),
    )(logits, logz)

    return log_prob


# speedup vs baseline: 1.2021x; 1.0460x over previous
import functools

import jax
import jax.numpy as jnp
from jax import lax
from jax.experimental import pallas as pl
from jax.experimental.pallas import tpu as pltpu
from jax.experimental.pallas import tpu_sc as plsc

_VOCAB = 1000000
_EMBED = 64
_CTX = 200
_HID = 120

_TILE = 32768
_NT = (_VOCAB + _TILE - 1) // _TILE
_VPAD = _NT * _TILE

_LANES = 128
_TAIL0 = (_VOCAB // _LANES) * _LANES
_TAILW = _VOCAB - _TAIL0

_NC = 2
_NS = 16
_RPW = 8
_ACTIVE = _CTX // _RPW


@functools.partial(
    pl.kernel,
    out_type=jax.ShapeDtypeStruct((_CTX, _EMBED, _LANES), jnp.float32),
    mesh=plsc.VectorSubcoreMesh(core_axis_name="c", subcore_axis_name="s"),
    scratch_types=[
        pltpu.VMEM((16,), jnp.int32),
        pltpu.VMEM((_RPW, _EMBED, _LANES), jnp.float32),
        pltpu.SemaphoreType.DMA,
    ],
)
def _sc_gather(x_hbm, embt_hbm, out_hbm, idx_v, tbuf, sem):
    wid = lax.axis_index("s") * _NC + lax.axis_index("c")

    @pl.when(wid < _ACTIVE)
    def _():
        base = pl.multiple_of(wid * _RPW, _RPW)
        pltpu.sync_copy(x_hbm.at[pl.ds(base, _RPW)], idx_v.at[pl.ds(0, _RPW)])
        idx_reg = idx_v[...]
        copies = []
        for j in range(_RPW):
            win = jnp.minimum(idx_reg[j] // _LANES, _TAIL0 // _LANES - 1)
            start = pl.multiple_of(win * _LANES, _LANES)
            copies.append(pltpu.async_copy(
                embt_hbm.at[:, pl.ds(start, _LANES)], tbuf.at[j], sem))
        for c in copies:
            c.wait()
        pltpu.sync_copy(tbuf, out_hbm.at[pl.ds(base, _RPW)])


def _mlp_body(win_ref, l_ref, tv_ref, w1_ref, b1_ref, w2t_ref, b2_ref,
              out_ref, h_s, m_s, s_s, lg_s):
    i = pl.program_id(0)

    @pl.when(i == 0)
    def _():
        lanes = lax.broadcasted_iota(jnp.int32, (_CTX, _EMBED, _LANES), 2)
        l3 = l_ref[...].reshape(_CTX, 1, 1)
        sel = jnp.where(lanes == l3, win_ref[...], 0.0)
        sel = jnp.where((l3 == _LANES) & (lanes == 0),
                        tv_ref[...][:, :, None], sel)
        wm = sel.reshape(_CTX * _EMBED, _LANES)
        h128 = lax.dot_general(
            w1_ref[...], wm, (((1,), (0,)), ((), ())),
            preferred_element_type=jnp.float32)
        hv = jnp.sum(h128, axis=1, keepdims=True)
        h_s[...] = jnp.maximum(hv + b1_ref[...], 0.0)
        m_s[...] = jnp.full((1, 128), -jnp.inf, jnp.float32)
        s_s[...] = jnp.zeros((1, 128), jnp.float32)

    @pl.when(i < _NT)
    def _():
        logits = lax.dot_general(
            h_s[...].astype(jnp.bfloat16), w2t_ref[...].astype(jnp.bfloat16),
            (((0,), (0,)), ((), ())),
            preferred_element_type=jnp.float32)
        logits = logits + b2_ref[...].reshape(1, _TILE)
        lg_s[pl.ds(i, 1), :] = logits

        lane = lax.broadcasted_iota(jnp.int32, (1, _TILE), 1)
        valid = lane < (_VOCAB - i * _TILE)
        lm = jnp.where(valid, logits, -jnp.inf)

        t_max = jnp.max(lm, axis=1, keepdims=True)
        m_old = m_s[0:1, 0:1]
        s_old = s_s[0:1, 0:1]
        m_new = jnp.maximum(m_old, t_max)
        t_sum = jnp.sum(jnp.exp(lm - m_new), axis=1, keepdims=True)
        s_new = s_old * jnp.exp(m_old - m_new) + t_sum
        m_s[0:1, 0:1] = m_new
        s_s[0:1, 0:1] = s_new

    @pl.when(i >= _NT)
    def _():
        logz = m_s[0:1, 0:1] + jnp.log(s_s[0:1, 0:1])
        out_ref[...] = lg_s[pl.ds(i - _NT, 1), :] - logz


def kernel(x, emb, W1, b1, W2, b2):
    embt = jnp.swapaxes(emb, 0, 1)
    xi = x.astype(jnp.int32)
    win = _sc_gather(xi, embt)
    cond = xi >= _TAIL0
    tail = lax.slice_in_dim(emb, _TAIL0, _VOCAB, axis=0)
    tv = jnp.take(tail, jnp.clip(xi - _TAIL0, 0, _TAILW - 1), axis=0)
    l_col = jnp.where(cond, _LANES, xi % _LANES).reshape(_CTX, 1)
    b1c = b1.reshape(_HID, 1)
    w2t = jnp.swapaxes(W2, 0, 1)

    log_prob = pl.pallas_call(
        _mlp_body,
        grid=(2 * _NT,),
        in_specs=[
            pl.BlockSpec((_CTX, _EMBED, _LANES), lambda i: (0, 0, 0)),
            pl.BlockSpec((_CTX, 1), lambda i: (0, 0)),
            pl.BlockSpec((_CTX, _EMBED), lambda i: (0, 0)),
            pl.BlockSpec((_HID, _CTX * _EMBED), lambda i: (0, 0)),
            pl.BlockSpec((_HID, 1), lambda i: (0, 0)),
            pl.BlockSpec((_HID, _TILE),
                         lambda i: (0, jnp.minimum(i, _NT - 1))),
            pl.BlockSpec((_TILE,), lambda i: (jnp.minimum(i, _NT - 1),)),
        ],
        out_specs=pl.BlockSpec(
            (1, _TILE), lambda i: (0, jnp.where(i < _NT, 0, i - _NT))),
        out_shape=jax.ShapeDtypeStruct((1, _VOCAB), jnp.float32),
        scratch_shapes=[
            pltpu.VMEM((_HID, 1), jnp.float32),
            pltpu.VMEM((1, 128), jnp.float32),
            pltpu.VMEM((1, 128), jnp.float32),
            pltpu.VMEM((_NT, _TILE), jnp.float32),
        ],
        compiler_params=pltpu.CompilerParams(
            dimension_semantics=("arbitrary",)),
    )(win, l_col, tv, W1, b1c, w2t, b2)

    return log_prob
